# baseline (speedup 1.0000x reference)
import functools
import jax
import jax.numpy as jnp
from jax import lax
from jax.experimental import pallas as pl
from jax.experimental.pallas import tpu as pltpu
from jax.experimental.pallas import tpu_sc as plsc

D = 64
SCALE = 8.0
L = 16

_info = plsc.get_sparse_core_info()
NC, NS = _info.num_cores, _info.num_subcores
NW = NC * NS

CHUNK = 512


def _make_emb(B):
    assert B % NW == 0
    b_per_w = B // NW
    assert b_per_w % CHUNK == 0
    n_chunks = b_per_w // CHUNK
    mesh = plsc.VectorSubcoreMesh(core_axis_name="c", subcore_axis_name="s")

    @functools.partial(
        pl.kernel, mesh=mesh,
        out_type=jax.ShapeDtypeStruct((B, D), jnp.float32),
        compiler_params=pltpu.CompilerParams(use_tc_tiling_on_sc=False),
        scratch_types=[
            pltpu.VMEM((CHUNK,), jnp.int32),
            pltpu.VMEM((CHUNK, D), jnp.float32),
            pltpu.SemaphoreType.DMA,
        ],
    )
    def _emb(idx_hbm, table_hbm, out_hbm, idx_v, rows_v, sem):
        wid = lax.axis_index("s") * NC + lax.axis_index("c")
        base = wid * b_per_w

        def chunk_body(i, _):
            off = base + i * CHUNK
            pltpu.sync_copy(idx_hbm.at[pl.ds(off, CHUNK)], idx_v)
            pltpu.async_copy(table_hbm.at[idx_v], rows_v, sem).wait()

            def scale_row(r, _):
                for c in range(D // L):
                    sl = (r, pl.ds(c * L, L))
                    rows_v[sl] = rows_v[sl] * SCALE
                return 0

            lax.fori_loop(0, CHUNK, scale_row, 0)
            pltpu.sync_copy(rows_v, out_hbm.at[pl.ds(off, CHUNK)])
            return 0

        lax.fori_loop(0, n_chunks, chunk_body, 0)

    return _emb


def kernel(x, table):
    orig_shape = x.shape
    idx = x.reshape(-1).astype(jnp.int32)
    out = _make_emb(idx.shape[0])(idx, table)
    return out.reshape(*orig_shape, D)


# speedup vs baseline: 1.1032x; 1.1032x over previous
import functools
import jax
import jax.numpy as jnp
from jax import lax
from jax.experimental import pallas as pl
from jax.experimental.pallas import tpu as pltpu
from jax.experimental.pallas import tpu_sc as plsc

D = 64
SCALE = 8.0
L = 16

_info = plsc.get_sparse_core_info()
NC, NS = _info.num_cores, _info.num_subcores
NW = NC * NS

CHUNK = 512
NBUF = 2


def _make_emb(B):
    assert B % NW == 0
    b_per_w = B // NW
    assert b_per_w % (NBUF * CHUNK) == 0
    n_chunks = b_per_w // CHUNK
    n_outer = n_chunks // NBUF
    mesh = plsc.VectorSubcoreMesh(core_axis_name="c", subcore_axis_name="s")

    @functools.partial(
        pl.kernel, mesh=mesh,
        out_type=jax.ShapeDtypeStruct((B, D), jnp.float32),
        compiler_params=pltpu.CompilerParams(use_tc_tiling_on_sc=False),
        scratch_types=[
            pltpu.VMEM((b_per_w,), jnp.int32),
            pltpu.VMEM((NBUF, CHUNK, D), jnp.float32),
            pltpu.SemaphoreType.DMA,
            pltpu.SemaphoreType.DMA,
            pltpu.SemaphoreType.DMA,
            pltpu.SemaphoreType.DMA,
        ],
    )
    def _emb(idx_hbm, table_hbm, out_hbm, idx_v, rows_v, g0, g1, o0, o1):
        gsem = [g0, g1]
        osem = [o0, o1]
        wid = lax.axis_index("s") * NC + lax.axis_index("c")
        base = wid * b_per_w
        pltpu.sync_copy(idx_hbm.at[pl.ds(base, b_per_w)], idx_v)

        def g_desc(c, b):
            return pltpu.make_async_copy(
                table_hbm.at[idx_v.at[pl.ds(c * CHUNK, CHUNK)]],
                rows_v.at[b], gsem[b])

        def o_desc(c, b):
            return pltpu.make_async_copy(
                rows_v.at[b], out_hbm.at[pl.ds(base + c * CHUNK, CHUNK)],
                osem[b])

        def scale(b):
            def row(r, _):
                for c in range(D // L):
                    sl = (r, pl.ds(c * L, L))
                    rows_v[b, sl[0], sl[1]] = rows_v[b, sl[0], sl[1]] * SCALE
                return 0
            lax.fori_loop(0, CHUNK, row, 0)

        g_desc(0, 0).start()

        def outer(o, _):
            for b in range(NBUF):
                c = o * NBUF + b
                nb = (b + 1) % NBUF
                @pl.when(c + 1 < n_chunks)
                def _():
                    @pl.when(c >= 1)
                    def _():
                        o_desc(c - 1, nb).wait()
                    g_desc(c + 1, nb).start()

                g_desc(c, b).wait()
                scale(b)
                o_desc(c, b).start()
            return 0

        lax.fori_loop(0, n_outer, outer, 0)
        o_desc(n_chunks - 2, (n_chunks - 2) % NBUF).wait()
        o_desc(n_chunks - 1, (n_chunks - 1) % NBUF).wait()

    return _emb


def kernel(x, table):
    orig_shape = x.shape
    idx = x.reshape(-1).astype(jnp.int32)
    out = _make_emb(idx.shape[0])(idx, table)
    return out.reshape(*orig_shape, D)
